# baseline (speedup 1.0000x reference)
import jax
import jax.numpy as jnp
from jax.experimental import pallas as pl
from jax.experimental.pallas import tpu as pltpu

_SLOPE = (1.0 / 8.0 + 1.0 / 3.0) / 2.0

_CHUNK = 2000


def _fused_kernel(x_hbm, w_ref, o_hbm, xbuf, obuf, insem, outsem):
    n = x_hbm.shape[0]
    d = x_hbm.shape[1]
    nchunks = n // _CHUNK

    def in_copy(i, slot):
        return pltpu.make_async_copy(
            x_hbm.at[pl.ds(i * _CHUNK, _CHUNK), :], xbuf.at[slot],
            insem.at[slot])

    def out_copy(i, slot):
        return pltpu.make_async_copy(
            obuf.at[slot], o_hbm.at[pl.ds(i * _CHUNK, _CHUNK), :],
            outsem.at[slot])

    in_copy(0, 0).start()
    in_copy(1, 1).start()
    for i in range(nchunks):
        slot = i % 2
        in_copy(i, slot).wait()
        x = xbuf[slot]
        h1 = jnp.where(x >= 0, x, x * _SLOPE)
        h2 = jnp.where(x >= 0, x, x * (_SLOPE * _SLOPE))
        res = (
            jnp.dot(h1, w_ref[:d], preferred_element_type=jnp.float32)
            + jnp.dot(h2, w_ref[d:2 * d], preferred_element_type=jnp.float32)
            + w_ref[2 * d:2 * d + 1]
        )
        if i >= 2:
            out_copy(i - 2, slot).wait()
        obuf[slot] = res
        out_copy(i, slot).start()
        if i + 2 < nchunks:
            in_copy(i + 2, slot).start()
    out_copy(nchunks - 2, (nchunks - 2) % 2).wait()
    out_copy(nchunks - 1, (nchunks - 1) % 2).wait()


def kernel(node_feats, edge_feats, edge_index, Wn0, bn0, Wl0, bl0,
           Wn1, bn1, Wl1, bl1, Wo, bo):
    n, d = node_feats.shape
    h = Wo.shape[1]
    w_packed = jnp.concatenate([Wo, bo.reshape(1, h),
                                jnp.zeros((7, h), Wo.dtype)], axis=0)
    return pl.pallas_call(
        _fused_kernel,
        in_specs=[
            pl.BlockSpec(memory_space=pl.ANY),
            pl.BlockSpec(memory_space=pltpu.VMEM),
        ],
        out_specs=pl.BlockSpec(memory_space=pl.ANY),
        out_shape=jax.ShapeDtypeStruct((n, h), jnp.float32),
        scratch_shapes=[
            pltpu.VMEM((2, _CHUNK, d), jnp.float32),
            pltpu.VMEM((2, _CHUNK, h), jnp.float32),
            pltpu.SemaphoreType.DMA((2,)),
            pltpu.SemaphoreType.DMA((2,)),
        ],
    )(node_feats, w_packed)


# speedup vs baseline: 1.2379x; 1.2379x over previous
import jax
import jax.numpy as jnp
from jax.experimental import pallas as pl
from jax.experimental.pallas import tpu as pltpu

_SLOPE = (1.0 / 8.0 + 1.0 / 3.0) / 2.0


def _fused_kernel(x_ref, w_ref, o_ref):
    d = x_ref.shape[-1]
    wt = w_ref[:d]
    wb = w_ref[d:2 * d]
    bias = w_ref[2 * d:2 * d + 1]
    b_mat = _SLOPE * wt + (_SLOPE * _SLOPE) * wb
    c_mat = (1.0 - _SLOPE) * wt + (1.0 - _SLOPE * _SLOPE) * wb
    x = x_ref[...]
    p = jnp.maximum(x, 0.0)
    o_ref[...] = (
        jnp.dot(x, b_mat, preferred_element_type=jnp.float32)
        + jnp.dot(p, c_mat, preferred_element_type=jnp.float32)
        + bias
    )


def kernel(node_feats, edge_feats, edge_index, Wn0, bn0, Wl0, bl0,
           Wn1, bn1, Wl1, bl1, Wo, bo):
    n, d = node_feats.shape
    h = Wo.shape[1]
    w_packed = jnp.concatenate([Wo, bo.reshape(1, h),
                                jnp.zeros((7, h), Wo.dtype)], axis=0)
    return pl.pallas_call(
        _fused_kernel,
        in_specs=[
            pl.BlockSpec(memory_space=pltpu.VMEM),
            pl.BlockSpec(memory_space=pltpu.VMEM),
        ],
        out_specs=pl.BlockSpec(memory_space=pltpu.VMEM),
        out_shape=jax.ShapeDtypeStruct((n, h), jnp.float32),
    )(node_feats, w_packed)
